# baseline (speedup 1.0000x reference)
import functools

import jax
import jax.numpy as jnp
from jax import lax
from jax.experimental import pallas as pl
from jax.experimental.pallas import tpu as pltpu
from jax.experimental.pallas import tpu_sc as plsc

_LANES = 16
_UNROLL = 16


def kernel(patch, pos_table):
    B, P, D = patch.shape
    info = plsc.get_sparse_core_info()
    NW = info.num_cores * info.num_subcores
    rows_w = P // NW
    chunk = rows_w * D

    mesh = plsc.VectorSubcoreMesh(core_axis_name="c", subcore_axis_name="s")

    @functools.partial(
        pl.kernel,
        mesh=mesh,
        out_type=jax.ShapeDtypeStruct((B * P * D,), jnp.float32),
        scratch_types=[
            pltpu.VMEM((chunk,), jnp.float32),
            pltpu.VMEM((chunk,), jnp.float32),
            pltpu.SemaphoreType.DMA,
        ],
    )
    def sc_add(patch_hbm, pos_hbm, out_hbm, pos_v, buf, sem):
        wid = lax.axis_index("s") * info.num_cores + lax.axis_index("c")
        pos_base = wid * chunk
        pltpu.sync_copy(pos_hbm.at[pl.ds(pos_base, chunk)], pos_v)

        n_vec = chunk // (_LANES * _UNROLL)

        def batch_body(b, carry):
            base = b * (P * D) + pos_base
            pltpu.sync_copy(patch_hbm.at[pl.ds(base, chunk)], buf)

            def add_body(i, c):
                off = i * (_LANES * _UNROLL)
                for u in range(_UNROLL):
                    sl = pl.ds(off + u * _LANES, _LANES)
                    plsc.addupdate(buf.at[sl], pos_v[sl])
                return c

            lax.fori_loop(0, n_vec, add_body, 0)
            pltpu.sync_copy(buf, out_hbm.at[pl.ds(base, chunk)])
            return carry

        lax.fori_loop(0, B, batch_body, 0)

    out = sc_add(patch.reshape(-1), pos_table.reshape(-1))
    return out.reshape(B, P, D)


# speedup vs baseline: 1.1898x; 1.1898x over previous
import functools

import jax
import jax.numpy as jnp
from jax import lax
from jax.experimental import pallas as pl
from jax.experimental.pallas import tpu as pltpu
from jax.experimental.pallas import tpu_sc as plsc

_LANES = 16
_UNROLL = 16


def kernel(patch, pos_table):
    B, P, D = patch.shape
    info = plsc.get_sparse_core_info()
    NW = info.num_cores * info.num_subcores
    rows_w = P // NW
    chunk = rows_w * D

    mesh = plsc.VectorSubcoreMesh(core_axis_name="c", subcore_axis_name="s")

    @functools.partial(
        pl.kernel,
        mesh=mesh,
        out_type=jax.ShapeDtypeStruct((B * P * D,), jnp.float32),
        scratch_types=[
            pltpu.VMEM((chunk,), jnp.float32),
            pltpu.VMEM((chunk,), jnp.float32),
            pltpu.VMEM((chunk,), jnp.float32),
            pltpu.SemaphoreType.DMA,
            pltpu.SemaphoreType.DMA,
            pltpu.SemaphoreType.DMA,
            pltpu.SemaphoreType.DMA,
        ],
    )
    def sc_add(patch_hbm, pos_hbm, out_hbm, pos_v, buf0, buf1,
               isem0, isem1, osem0, osem1):
        wid = lax.axis_index("s") * info.num_cores + lax.axis_index("c")
        pos_base = wid * chunk
        pltpu.sync_copy(pos_hbm.at[pl.ds(pos_base, chunk)], pos_v)

        bufs = (buf0, buf1)
        isems = (isem0, isem1)
        osems = (osem0, osem1)

        def start_in(b, s):
            pltpu.async_copy(
                patch_hbm.at[pl.ds(b * (P * D) + pos_base, chunk)],
                bufs[s], isems[s])

        def wait_in(b, s):
            pltpu.make_async_copy(
                patch_hbm.at[pl.ds(b * (P * D) + pos_base, chunk)],
                bufs[s], isems[s]).wait()

        def start_out(b, s):
            pltpu.async_copy(
                bufs[s],
                out_hbm.at[pl.ds(b * (P * D) + pos_base, chunk)], osems[s])

        def wait_out(b, s):
            pltpu.make_async_copy(
                bufs[s],
                out_hbm.at[pl.ds(b * (P * D) + pos_base, chunk)], osems[s]).wait()

        n_vec = chunk // (_LANES * _UNROLL)

        def add_pos(s):
            buf = bufs[s]

            def add_body(i, c):
                off = i * (_LANES * _UNROLL)
                for u in range(_UNROLL):
                    sl = pl.ds(off + u * _LANES, _LANES)
                    plsc.addupdate(buf.at[sl], pos_v[sl])
                return c

            lax.fori_loop(0, n_vec, add_body, 0)

        start_in(0, 0)

        def pair_body(k, carry):
            for s in range(2):
                b = 2 * k + s
                o = 1 - s
                @pl.when(b >= 1)
                def _():
                    wait_out(b - 1, o)

                @pl.when(b + 1 < B)
                def _():
                    start_in(b + 1, o)

                wait_in(b, s)
                add_pos(s)
                start_out(b, s)
            return carry

        lax.fori_loop(0, B // 2, pair_body, 0)
        wait_out(B - 1, 1)

    out = sc_add(patch.reshape(-1), pos_table.reshape(-1))
    return out.reshape(B, P, D)


# speedup vs baseline: 1.1907x; 1.0007x over previous
import functools

import jax
import jax.numpy as jnp
from jax import lax
from jax.experimental import pallas as pl
from jax.experimental.pallas import tpu as pltpu
from jax.experimental.pallas import tpu_sc as plsc

_LANES = 16
_UNROLL = 8


def kernel(patch, pos_table):
    B, P, D = patch.shape
    info = plsc.get_sparse_core_info()
    NW = info.num_cores * info.num_subcores
    rows_w = P // NW
    chunk = rows_w * D

    mesh = plsc.VectorSubcoreMesh(core_axis_name="c", subcore_axis_name="s")

    @functools.partial(
        pl.kernel,
        mesh=mesh,
        out_type=jax.ShapeDtypeStruct((B * P * D,), jnp.float32),
        scratch_types=[
            pltpu.VMEM((chunk,), jnp.float32),
            pltpu.VMEM((chunk,), jnp.float32),
            pltpu.VMEM((chunk,), jnp.float32),
            pltpu.SemaphoreType.DMA,
            pltpu.SemaphoreType.DMA,
            pltpu.SemaphoreType.DMA,
            pltpu.SemaphoreType.DMA,
        ],
    )
    def sc_add(patch_hbm, pos_hbm, out_hbm, pos_v, buf0, buf1,
               isem0, isem1, osem0, osem1):
        wid = lax.axis_index("s") * info.num_cores + lax.axis_index("c")
        pos_base = wid * chunk
        pltpu.sync_copy(pos_hbm.at[pl.ds(pos_base, chunk)], pos_v)

        bufs = (buf0, buf1)
        isems = (isem0, isem1)
        osems = (osem0, osem1)

        def start_in(b, s):
            pltpu.async_copy(
                patch_hbm.at[pl.ds(b * (P * D) + pos_base, chunk)],
                bufs[s], isems[s])

        def wait_in(b, s):
            pltpu.make_async_copy(
                patch_hbm.at[pl.ds(b * (P * D) + pos_base, chunk)],
                bufs[s], isems[s]).wait()

        def start_out(b, s):
            pltpu.async_copy(
                bufs[s],
                out_hbm.at[pl.ds(b * (P * D) + pos_base, chunk)], osems[s])

        def wait_out(b, s):
            pltpu.make_async_copy(
                bufs[s],
                out_hbm.at[pl.ds(b * (P * D) + pos_base, chunk)], osems[s]).wait()

        def add_pos(s):
            buf = bufs[s]

            @plsc.parallel_loop(0, chunk // _LANES, unroll=_UNROLL)
            def _(i):
                sl = pl.ds(i * _LANES, _LANES)
                plsc.addupdate(buf.at[sl], pos_v[sl])

        start_in(0, 0)

        def pair_body(k, carry):
            for s in range(2):
                b = 2 * k + s
                o = 1 - s
                @pl.when(b >= 1)
                def _():
                    wait_out(b - 1, o)

                @pl.when(b + 1 < B)
                def _():
                    start_in(b + 1, o)

                wait_in(b, s)
                add_pos(s)
                start_out(b, s)
            return carry

        lax.fori_loop(0, B // 2, pair_body, 0)
        wait_out(B - 1, 1)

    out = sc_add(patch.reshape(-1), pos_table.reshape(-1))
    return out.reshape(B, P, D)


# speedup vs baseline: 1.2664x; 1.0636x over previous
import functools

import jax
import jax.numpy as jnp
from jax import lax
from jax.experimental import pallas as pl
from jax.experimental.pallas import tpu as pltpu
from jax.experimental.pallas import tpu_sc as plsc

_LANES = 16
_UNROLL = 8


def kernel(patch, pos_table):
    B, P, D = patch.shape
    info = plsc.get_sparse_core_info()
    NW = info.num_cores * info.num_subcores
    rows_w = P // NW
    chunk = rows_w * D

    mesh = plsc.VectorSubcoreMesh(core_axis_name="c", subcore_axis_name="s")

    @functools.partial(
        pl.kernel,
        mesh=mesh,
        out_type=jax.ShapeDtypeStruct((B * P * D,), jnp.float32),
        scratch_types=[
            pltpu.VMEM((chunk,), jnp.float32),
            pltpu.VMEM((chunk,), jnp.float32),
            pltpu.VMEM((chunk,), jnp.float32),
            pltpu.VMEM((chunk,), jnp.float32),
            pltpu.VMEM((chunk,), jnp.float32),
            pltpu.SemaphoreType.DMA,
            pltpu.SemaphoreType.DMA,
            pltpu.SemaphoreType.DMA,
            pltpu.SemaphoreType.DMA,
            pltpu.SemaphoreType.DMA,
            pltpu.SemaphoreType.DMA,
            pltpu.SemaphoreType.DMA,
            pltpu.SemaphoreType.DMA,
        ],
    )
    def sc_add(patch_hbm, pos_hbm, out_hbm, pos_v, buf0, buf1, buf2, buf3,
               isem0, isem1, isem2, isem3, osem0, osem1, osem2, osem3):
        wid = lax.axis_index("s") * info.num_cores + lax.axis_index("c")
        pos_base = wid * chunk
        pltpu.sync_copy(pos_hbm.at[pl.ds(pos_base, chunk)], pos_v)

        bufs = (buf0, buf1, buf2, buf3)
        isems = (isem0, isem1, isem2, isem3)
        osems = (osem0, osem1, osem2, osem3)
        NS = 4

        def start_in(b, s):
            pltpu.async_copy(
                patch_hbm.at[pl.ds(b * (P * D) + pos_base, chunk)],
                bufs[s], isems[s])

        def wait_in(b, s):
            pltpu.make_async_copy(
                patch_hbm.at[pl.ds(b * (P * D) + pos_base, chunk)],
                bufs[s], isems[s]).wait()

        def start_out(b, s):
            pltpu.async_copy(
                bufs[s],
                out_hbm.at[pl.ds(b * (P * D) + pos_base, chunk)], osems[s])

        def wait_out(b, s):
            pltpu.make_async_copy(
                bufs[s],
                out_hbm.at[pl.ds(b * (P * D) + pos_base, chunk)], osems[s]).wait()

        def add_pos(s):
            buf = bufs[s]

            @plsc.parallel_loop(0, chunk // _LANES, unroll=_UNROLL)
            def _(i):
                sl = pl.ds(i * _LANES, _LANES)
                plsc.addupdate(buf.at[sl], pos_v[sl])

        start_in(0, 0)
        start_in(1, 1)

        def ring_body(k, carry):
            for s in range(NS):
                b = NS * k + s
                wait_in(b, s)
                add_pos(s)
                start_out(b, s)
                s2 = (s + 2) % NS

                @pl.when(b >= 2)
                def _():
                    wait_out(b - 2, s2)

                @pl.when(b + 2 < B)
                def _():
                    start_in(b + 2, s2)
            return carry

        lax.fori_loop(0, B // NS, ring_body, 0)
        wait_out(B - 2, (B - 2) % NS)
        wait_out(B - 1, (B - 1) % NS)

    out = sc_add(patch.reshape(-1), pos_table.reshape(-1))
    return out.reshape(B, P, D)


# speedup vs baseline: 1.3905x; 1.0980x over previous
import functools

import jax
import jax.numpy as jnp
from jax import lax
from jax.experimental import pallas as pl
from jax.experimental.pallas import tpu as pltpu
from jax.experimental.pallas import tpu_sc as plsc

_LANES = 16
_UNROLL = 8
_TC_BATCHES = 52


def _add_body(patch_ref, pos_ref, out_ref):
    out_ref[...] = patch_ref[...] + pos_ref[...]


def _tc_part(patch, pos_table, nb):
    B, P, D = patch.shape
    return pl.pallas_call(
        _add_body,
        grid=(nb,),
        in_specs=[
            pl.BlockSpec((1, P, D), lambda b: (b, 0, 0)),
            pl.BlockSpec((P, D), lambda b: (0, 0)),
        ],
        out_specs=pl.BlockSpec((1, P, D), lambda b: (b, 0, 0)),
        out_shape=jax.ShapeDtypeStruct((nb, P, D), patch.dtype),
    )(patch, pos_table)


def _sc_part(patch_flat, pos_flat, B, P, D, b_lo, b_hi):
    info = plsc.get_sparse_core_info()
    NW = info.num_cores * info.num_subcores
    rows_w = P // NW
    chunk = rows_w * D
    nb = b_hi - b_lo

    mesh = plsc.VectorSubcoreMesh(core_axis_name="c", subcore_axis_name="s")

    @functools.partial(
        pl.kernel,
        mesh=mesh,
        out_type=jax.ShapeDtypeStruct((nb * P * D,), jnp.float32),
        scratch_types=[
            pltpu.VMEM((chunk,), jnp.float32),
            pltpu.VMEM((chunk,), jnp.float32),
            pltpu.VMEM((chunk,), jnp.float32),
            pltpu.VMEM((chunk,), jnp.float32),
            pltpu.VMEM((chunk,), jnp.float32),
            pltpu.SemaphoreType.DMA,
            pltpu.SemaphoreType.DMA,
            pltpu.SemaphoreType.DMA,
            pltpu.SemaphoreType.DMA,
            pltpu.SemaphoreType.DMA,
            pltpu.SemaphoreType.DMA,
            pltpu.SemaphoreType.DMA,
            pltpu.SemaphoreType.DMA,
        ],
    )
    def sc_add(patch_hbm, pos_hbm, out_hbm, pos_v, buf0, buf1, buf2, buf3,
               isem0, isem1, isem2, isem3, osem0, osem1, osem2, osem3):
        wid = lax.axis_index("s") * info.num_cores + lax.axis_index("c")
        pos_base = wid * chunk
        pltpu.sync_copy(pos_hbm.at[pl.ds(pos_base, chunk)], pos_v)

        bufs = (buf0, buf1, buf2, buf3)
        isems = (isem0, isem1, isem2, isem3)
        osems = (osem0, osem1, osem2, osem3)
        NS = 4

        def start_in(j, s):
            pltpu.async_copy(
                patch_hbm.at[pl.ds((b_lo + j) * (P * D) + pos_base, chunk)],
                bufs[s], isems[s])

        def wait_in(j, s):
            pltpu.make_async_copy(
                patch_hbm.at[pl.ds((b_lo + j) * (P * D) + pos_base, chunk)],
                bufs[s], isems[s]).wait()

        def start_out(j, s):
            pltpu.async_copy(
                bufs[s],
                out_hbm.at[pl.ds(j * (P * D) + pos_base, chunk)], osems[s])

        def wait_out(j, s):
            pltpu.make_async_copy(
                bufs[s],
                out_hbm.at[pl.ds(j * (P * D) + pos_base, chunk)], osems[s]).wait()

        def add_pos(s):
            buf = bufs[s]

            @plsc.parallel_loop(0, chunk // _LANES, unroll=_UNROLL)
            def _(i):
                sl = pl.ds(i * _LANES, _LANES)
                plsc.addupdate(buf.at[sl], pos_v[sl])

        start_in(0, 0)
        start_in(1, 1)

        def ring_body(k, carry):
            for s in range(NS):
                j = NS * k + s
                wait_in(j, s)
                add_pos(s)
                start_out(j, s)
                s2 = (s + 2) % NS

                @pl.when(j >= 2)
                def _():
                    wait_out(j - 2, s2)

                @pl.when(j + 2 < nb)
                def _():
                    start_in(j + 2, s2)
            return carry

        lax.fori_loop(0, nb // NS, ring_body, 0)
        wait_out(nb - 2, (nb - 2) % NS)
        wait_out(nb - 1, (nb - 1) % NS)

    return sc_add(patch_flat, pos_flat)


def kernel(patch, pos_table):
    B, P, D = patch.shape
    nb_tc = _TC_BATCHES
    sc_out = _sc_part(patch.reshape(-1), pos_table.reshape(-1),
                      B, P, D, nb_tc, B)
    tc_out = _tc_part(patch, pos_table, nb_tc)
    return jnp.concatenate(
        [tc_out, sc_out.reshape(B - nb_tc, P, D)], axis=0)


# speedup vs baseline: 5.5828x; 4.0148x over previous
import jax
import jax.numpy as jnp
from jax.experimental import pallas as pl


def _add_body(patch_ref, pos_ref, out_ref):
    out_ref[...] = patch_ref[...] + pos_ref[...]


def kernel(patch, pos_table):
    B, P, D = patch.shape
    return pl.pallas_call(
        _add_body,
        grid=(B // 2,),
        in_specs=[
            pl.BlockSpec((2, P, D), lambda b: (b, 0, 0)),
            pl.BlockSpec((P, D), lambda b: (0, 0)),
        ],
        out_specs=pl.BlockSpec((2, P, D), lambda b: (b, 0, 0)),
        out_shape=jax.ShapeDtypeStruct(patch.shape, patch.dtype),
    )(patch, pos_table)


# speedup vs baseline: 5.6378x; 1.0099x over previous
import jax
import jax.numpy as jnp
from jax.experimental import pallas as pl


def _add_body(patch_ref, pos_ref, out_ref):
    out_ref[...] = patch_ref[...] + pos_ref[...]


def kernel(patch, pos_table):
    B, P, D = patch.shape
    return pl.pallas_call(
        _add_body,
        grid=(B // 4,),
        in_specs=[
            pl.BlockSpec((4, P, D), lambda b: (b, 0, 0)),
            pl.BlockSpec((P, D), lambda b: (0, 0)),
        ],
        out_specs=pl.BlockSpec((4, P, D), lambda b: (b, 0, 0)),
        out_shape=jax.ShapeDtypeStruct(patch.shape, patch.dtype),
    )(patch, pos_table)
